# baseline (speedup 1.0000x reference)
import functools

import jax
import jax.numpy as jnp
from jax import lax
from jax.experimental import pallas as pl
from jax.experimental.pallas import tpu as pltpu
from jax.experimental.pallas import tpu_sc as plsc

D = 64
DP = 128
T = 4096
B = 4
V = 256
ROWS = B * T
PAIRS = ROWS // 2
NC, NS, L = 2, 16, 16
NW = NC * NS
PPW = PAIRS // NW

BLK2 = 2048


_MESH = plsc.VectorSubcoreMesh(core_axis_name="c", subcore_axis_name="s")


@functools.partial(
    pl.kernel,
    out_type=jax.ShapeDtypeStruct((PAIRS, DP), jnp.float32),
    mesh=_MESH,
    compiler_params=pltpu.CompilerParams(use_tc_tiling_on_sc=False),
    scratch_types=[
        pltpu.VMEM((PPW,), jnp.int32),
        pltpu.VMEM((PPW,), jnp.int32),
        pltpu.VMEM((PPW, D), jnp.float32),
        pltpu.VMEM((PPW, D), jnp.float32),
        pltpu.SemaphoreType.DMA,
        pltpu.SemaphoreType.DMA,
    ],
)
def _sc_gather(xe_hbm, xo_hbm, table_hbm, out_hbm, idxe_v, idxo_v,
               rowse_v, rowso_v, seme, semo):
    wid = lax.axis_index("s") * NC + lax.axis_index("c")
    base = wid * PPW
    pltpu.sync_copy(xe_hbm.at[pl.ds(base, PPW)], idxe_v)
    pltpu.sync_copy(xo_hbm.at[pl.ds(base, PPW)], idxo_v)
    ge = pltpu.async_copy(table_hbm.at[idxe_v], rowse_v, seme)
    go = pltpu.async_copy(table_hbm.at[idxo_v], rowso_v, semo)
    out_slice = out_hbm.at[pl.ds(base, PPW)]
    ge.wait()
    pltpu.sync_copy(rowse_v, out_slice.at[:, pl.ds(0, D)])
    go.wait()
    pltpu.sync_copy(rowso_v, out_slice.at[:, pl.ds(D, D)])


def _finish_body(g_ref, pos_ref, w_ref, b_ref, y_ref):
    h = g_ref[...] + pos_ref[...]
    dn = (((1,), (1,)), ((), ()))
    w = w_ref[...]
    ye = lax.dot_general(h[:, :D], w, dn, preferred_element_type=jnp.float32)
    yo = lax.dot_general(h[:, D:], w, dn, preferred_element_type=jnp.float32)
    y_ref[...] = jnp.concatenate([ye, yo], axis=1) + b_ref[...]


def _tc_finish(g2, pos2, W, b2):
    nblk = PAIRS // BLK2
    return pl.pallas_call(
        _finish_body,
        grid=(nblk,),
        in_specs=[
            pl.BlockSpec((BLK2, DP), lambda i: (i, 0)),
            pl.BlockSpec((BLK2, DP), lambda i: (i % (T // (2 * BLK2) or 1), 0)),
            pl.BlockSpec((D, D), lambda i: (0, 0)),
            pl.BlockSpec((1, DP), lambda i: (0, 0)),
        ],
        out_specs=pl.BlockSpec((BLK2, DP), lambda i: (i, 0)),
        out_shape=jax.ShapeDtypeStruct((PAIRS, DP), jnp.float32),
    )(g2, pos2, W, b2)


def kernel(x, byte_embed, pos_embed, W, b):
    x_flat = x.reshape(ROWS).astype(jnp.int32)
    xe = x_flat[0::2]
    xo = x_flat[1::2]
    g2 = _sc_gather(xe, xo, byte_embed)
    pos2 = pos_embed.reshape(T // 2, DP)
    b2 = jnp.concatenate([b, b]).reshape(1, DP)
    y2 = _tc_finish(g2, pos2, W, b2)
    return y2.reshape(B, T, D)


# speedup vs baseline: 1.0432x; 1.0432x over previous
import functools

import jax
import jax.numpy as jnp
from jax import lax
from jax.experimental import pallas as pl
from jax.experimental.pallas import tpu as pltpu
from jax.experimental.pallas import tpu_sc as plsc

D = 64
DP = 128
T = 4096
B = 4
V = 256
ROWS = B * T
HALF = ROWS // 2
NC, NS, L = 2, 16, 16
NW = NC * NS
PPW = HALF // NW

BLK = 2048


_MESH = plsc.VectorSubcoreMesh(core_axis_name="c", subcore_axis_name="s")


@functools.partial(
    pl.kernel,
    out_type=jax.ShapeDtypeStruct((HALF, DP), jnp.float32),
    mesh=_MESH,
    compiler_params=pltpu.CompilerParams(use_tc_tiling_on_sc=False),
    scratch_types=[
        pltpu.VMEM((PPW,), jnp.int32),
        pltpu.VMEM((PPW,), jnp.int32),
        pltpu.VMEM((PPW, D), jnp.float32),
        pltpu.VMEM((PPW, D), jnp.float32),
        pltpu.SemaphoreType.DMA,
        pltpu.SemaphoreType.DMA,
    ],
)
def _sc_gather(x_hbm, table_hbm, out_hbm, idxa_v, idxb_v, rowsa_v, rowsb_v,
               sema, semb):
    wid = lax.axis_index("s") * NC + lax.axis_index("c")
    base = wid * PPW
    pltpu.sync_copy(x_hbm.at[pl.ds(base, PPW)], idxa_v)
    pltpu.sync_copy(x_hbm.at[pl.ds(base + HALF, PPW)], idxb_v)
    ga = pltpu.async_copy(table_hbm.at[idxa_v], rowsa_v, sema)
    gb = pltpu.async_copy(table_hbm.at[idxb_v], rowsb_v, semb)
    out_slice = out_hbm.at[pl.ds(base, PPW)]
    ga.wait()
    pltpu.sync_copy(rowsa_v, out_slice.at[:, pl.ds(0, D)])
    gb.wait()
    pltpu.sync_copy(rowsb_v, out_slice.at[:, pl.ds(D, D)])


def _finish_body(g_ref, pos_ref, w_ref, b_ref, ya_ref, yb_ref):
    pb = pos_ref[...]
    h2 = g_ref[...] + jnp.concatenate([pb, pb], axis=1)
    dn = (((1,), (1,)), ((), ()))
    w = w_ref[...]
    bias = b_ref[...]
    ya_ref[...] = lax.dot_general(h2[:, :D], w, dn,
                                  preferred_element_type=jnp.float32) + bias
    yb_ref[...] = lax.dot_general(h2[:, D:], w, dn,
                                  preferred_element_type=jnp.float32) + bias


def _tc_finish(g2, pos_embed, W, b2d):
    nblk = HALF // BLK
    nblk_t = T // BLK
    return pl.pallas_call(
        _finish_body,
        grid=(nblk,),
        in_specs=[
            pl.BlockSpec((BLK, DP), lambda i: (i, 0)),
            pl.BlockSpec((BLK, D), lambda i: (i % nblk_t, 0)),
            pl.BlockSpec((D, D), lambda i: (0, 0)),
            pl.BlockSpec((1, D), lambda i: (0, 0)),
        ],
        out_specs=[
            pl.BlockSpec((BLK, D), lambda i: (i, 0)),
            pl.BlockSpec((BLK, D), lambda i: (i, 0)),
        ],
        out_shape=[
            jax.ShapeDtypeStruct((HALF, D), jnp.float32),
            jax.ShapeDtypeStruct((HALF, D), jnp.float32),
        ],
    )(g2, pos_embed, W, b2d)


def kernel(x, byte_embed, pos_embed, W, b):
    x_flat = x.reshape(ROWS).astype(jnp.int32)
    g2 = _sc_gather(x_flat, byte_embed)
    ya, yb = _tc_finish(g2, pos_embed, W, b.reshape(1, D))
    return jnp.concatenate([ya, yb], axis=0).reshape(B, T, D)


# speedup vs baseline: 1.0627x; 1.0187x over previous
import functools

import jax
import jax.numpy as jnp
from jax import lax
from jax.experimental import pallas as pl
from jax.experimental.pallas import tpu as pltpu
from jax.experimental.pallas import tpu_sc as plsc

D = 64
DP = 128
T = 4096
TH = T // 2
B = 4
V = 256
ROWS = B * T
HALF = ROWS // 2
NC, NS, L = 2, 16, 16
NW = NC * NS
PPW = HALF // NW


_MESH = plsc.VectorSubcoreMesh(core_axis_name="c", subcore_axis_name="s")


@functools.partial(
    pl.kernel,
    out_type=jax.ShapeDtypeStruct((HALF, DP), jnp.float32),
    mesh=_MESH,
    compiler_params=pltpu.CompilerParams(use_tc_tiling_on_sc=False),
    scratch_types=[
        pltpu.VMEM((PPW,), jnp.int32),
        pltpu.VMEM((PPW,), jnp.int32),
        pltpu.VMEM((PPW, D), jnp.float32),
        pltpu.VMEM((PPW, D), jnp.float32),
        pltpu.SemaphoreType.DMA,
        pltpu.SemaphoreType.DMA,
    ],
)
def _sc_gather(x_hbm, table_hbm, out_hbm, idxa_v, idxb_v, rowsa_v, rowsb_v,
               sema, semb):
    wid = lax.axis_index("s") * NC + lax.axis_index("c")
    base = wid * PPW
    bb = wid // 8
    flata = bb * T + (wid % 8) * PPW
    pltpu.sync_copy(x_hbm.at[pl.ds(flata, PPW)], idxa_v)
    pltpu.sync_copy(x_hbm.at[pl.ds(flata + TH, PPW)], idxb_v)
    ga = pltpu.async_copy(table_hbm.at[idxa_v], rowsa_v, sema)
    gb = pltpu.async_copy(table_hbm.at[idxb_v], rowsb_v, semb)
    out_slice = out_hbm.at[pl.ds(base, PPW)]
    ga.wait()
    pltpu.sync_copy(rowsa_v, out_slice.at[:, pl.ds(0, D)])
    gb.wait()
    pltpu.sync_copy(rowsb_v, out_slice.at[:, pl.ds(D, D)])


def _finish_body(g_ref, pos_ref, w_ref, b_ref, y_ref):
    h = pl.program_id(0)
    pb = pos_ref[...]
    dn = (((1,), (1,)), ((), ()))
    w = w_ref[...]
    bias = b_ref[...]

    @pl.when(h == 0)
    def _():
        y_ref[0] = lax.dot_general(g_ref[:, :D] + pb, w, dn,
                                   preferred_element_type=jnp.float32) + bias

    @pl.when(h == 1)
    def _():
        y_ref[0] = lax.dot_general(g_ref[:, D:] + pb, w, dn,
                                   preferred_element_type=jnp.float32) + bias


def _tc_finish(g2, pos_embed, W, b2d):
    return pl.pallas_call(
        _finish_body,
        grid=(2, B),
        in_specs=[
            pl.BlockSpec((TH, DP), lambda h, bb: (bb, 0)),
            pl.BlockSpec((TH, D), lambda h, bb: (h, 0)),
            pl.BlockSpec((D, D), lambda h, bb: (0, 0)),
            pl.BlockSpec((1, D), lambda h, bb: (0, 0)),
        ],
        out_specs=pl.BlockSpec((1, TH, D), lambda h, bb: (bb, h, 0)),
        out_shape=jax.ShapeDtypeStruct((B, T, D), jnp.float32),
    )(g2, pos_embed, W, b2d)


def kernel(x, byte_embed, pos_embed, W, b):
    x_flat = x.reshape(ROWS).astype(jnp.int32)
    g2 = _sc_gather(x_flat, byte_embed)
    return _tc_finish(g2, pos_embed, W, b.reshape(1, D))


# speedup vs baseline: 1.1523x; 1.0843x over previous
import functools

import jax
import jax.numpy as jnp
from jax import lax
from jax.experimental import pallas as pl
from jax.experimental.pallas import tpu as pltpu
from jax.experimental.pallas import tpu_sc as plsc

D = 64
DP = 128
T = 4096
TH = T // 2
B = 4
V = 256
ROWS = B * T
HALF = ROWS // 2
NC, NS, L = 2, 16, 16
NW = NC * NS
PPW = HALF // NW


_MESH = plsc.VectorSubcoreMesh(core_axis_name="c", subcore_axis_name="s")


@functools.partial(
    pl.kernel,
    out_type=jax.ShapeDtypeStruct((HALF, DP), jnp.float32),
    mesh=_MESH,
    compiler_params=pltpu.CompilerParams(use_tc_tiling_on_sc=False),
    scratch_types=[
        pltpu.VMEM((PPW,), jnp.int32),
        pltpu.VMEM((PPW,), jnp.int32),
        pltpu.VMEM((PPW, D), jnp.float32),
        pltpu.VMEM((PPW, D), jnp.float32),
        pltpu.SemaphoreType.DMA,
        pltpu.SemaphoreType.DMA,
    ],
)
def _sc_gather(x_hbm, table_hbm, out_hbm, idxa_v, idxb_v, rowsa_v, rowsb_v,
               sema, semb):
    wid = lax.axis_index("s") * NC + lax.axis_index("c")
    base = wid * PPW
    bb = wid // 8
    flata = bb * T + (wid % 8) * PPW
    pltpu.sync_copy(x_hbm.at[pl.ds(flata, PPW)], idxa_v)
    pltpu.sync_copy(x_hbm.at[pl.ds(flata + TH, PPW)], idxb_v)
    ga = pltpu.async_copy(table_hbm.at[idxa_v], rowsa_v, sema)
    gb = pltpu.async_copy(table_hbm.at[idxb_v], rowsb_v, semb)
    out_slice = out_hbm.at[pl.ds(base, PPW)]
    ga.wait()
    pltpu.sync_copy(rowsa_v, out_slice.at[:, pl.ds(0, D)])
    gb.wait()
    pltpu.sync_copy(rowsb_v, out_slice.at[:, pl.ds(D, D)])


def _finish_body(g_ref, pos_ref, w_ref, b_ref, y_ref):
    dn = (((1,), (1,)), ((), ()))
    w = w_ref[...]
    bias = b_ref[...]
    y_ref[0, :TH] = lax.dot_general(g_ref[:, :D] + pos_ref[:TH], w, dn,
                                    preferred_element_type=jnp.float32) + bias
    y_ref[0, TH:] = lax.dot_general(g_ref[:, D:] + pos_ref[TH:], w, dn,
                                    preferred_element_type=jnp.float32) + bias


def _tc_finish(g2, pos_embed, W, b2d):
    return pl.pallas_call(
        _finish_body,
        grid=(B,),
        in_specs=[
            pl.BlockSpec((TH, DP), lambda bb: (bb, 0)),
            pl.BlockSpec((T, D), lambda bb: (0, 0)),
            pl.BlockSpec((D, D), lambda bb: (0, 0)),
            pl.BlockSpec((1, D), lambda bb: (0, 0)),
        ],
        out_specs=pl.BlockSpec((1, T, D), lambda bb: (bb, 0, 0)),
        out_shape=jax.ShapeDtypeStruct((B, T, D), jnp.float32),
    )(g2, pos_embed, W, b2d)


def kernel(x, byte_embed, pos_embed, W, b):
    x_flat = x.reshape(ROWS).astype(jnp.int32)
    g2 = _sc_gather(x_flat, byte_embed)
    return _tc_finish(g2, pos_embed, W, b.reshape(1, D))
